# baseline (speedup 1.0000x reference)
import functools

import jax
import jax.numpy as jnp
from jax import lax
from jax.experimental import pallas as pl
from jax.experimental.pallas import tpu as pltpu
from jax.experimental.pallas import tpu_sc as plsc

_TIME_SCALE = 30.0
_B, _H, _W = 4096, 224, 224

_INFO = plsc.get_sparse_core_info()
_NC, _NS, _L = _INFO.num_cores, _INFO.num_subcores, _INFO.num_lanes
_NW = _NC * _NS
_BPW = _B // _NW
_CHUNKS = _BPW // _L


def _tc_body(aps_smem, img_hbm, rows_hbm, rows_v, copy_sem, out_sem):
    def issue(b, carry):
        y = aps_smem[2 * b + 1]
        pltpu.make_async_copy(
            img_hbm.at[b, y, :], rows_v.at[b], copy_sem).start()
        return carry

    lax.fori_loop(0, _B, issue, 0)

    def drain(b, carry):
        pltpu.make_async_copy(
            img_hbm.at[b, 0, :], rows_v.at[b], copy_sem).wait()
        return carry

    lax.fori_loop(0, _B, drain, 0)

    cp = pltpu.make_async_copy(rows_v, rows_hbm, out_sem)
    cp.start()
    cp.wait()


_tc_gather_rows = pl.pallas_call(
    _tc_body,
    out_shape=jax.ShapeDtypeStruct((_B, _W), jnp.float32),
    in_specs=[
        pl.BlockSpec(memory_space=pltpu.SMEM),
        pl.BlockSpec(memory_space=pl.ANY),
    ],
    out_specs=pl.BlockSpec(memory_space=pl.ANY),
    scratch_shapes=[
        pltpu.VMEM((_B, _W), jnp.float32),
        pltpu.SemaphoreType.DMA,
        pltpu.SemaphoreType.DMA,
    ],
)

_mesh = plsc.VectorSubcoreMesh(core_axis_name="c", subcore_axis_name="s")


@functools.partial(
    pl.kernel,
    mesh=_mesh,
    compiler_params=pltpu.CompilerParams(needs_layout_passes=False),
    out_type=jax.ShapeDtypeStruct((_B,), jnp.float32),
    scratch_types=[
        pltpu.VMEM((_BPW,), jnp.int32),
        pltpu.VMEM((_BPW, _W), jnp.float32),
        pltpu.VMEM((_BPW,), jnp.float32),
    ],
)
def _sc_pick(rows_hbm, x_hbm, out_hbm, x_v, rows_v, vals_v):
    wid = lax.axis_index("s") * _NC + lax.axis_index("c")
    base = wid * _BPW

    pltpu.sync_copy(x_hbm.at[pl.ds(base, _BPW)], x_v)
    pltpu.sync_copy(rows_hbm.at[pl.ds(base, _BPW), :], rows_v)

    lane = lax.iota(jnp.int32, _L)
    for g in range(_CHUNKS):
        sl = pl.ds(_L * g, _L)
        picked = plsc.load_gather(rows_v, [g * _L + lane, x_v[sl]])
        vals_v[sl] = picked * _TIME_SCALE

    pltpu.sync_copy(vals_v, out_hbm.at[pl.ds(base, _BPW)])


def kernel(v_image, actor_pixel_selection):
    rows = _tc_gather_rows(actor_pixel_selection.reshape(-1), v_image)
    x = actor_pixel_selection[:, 0]
    out = _sc_pick(rows, x)
    return out.reshape(_B, 1, 1)


# speedup vs baseline: 41.7549x; 41.7549x over previous
import functools

import jax
import jax.numpy as jnp
from jax import lax
from jax.experimental import pallas as pl
from jax.experimental.pallas import tpu as pltpu
from jax.experimental.pallas import tpu_sc as plsc

_TIME_SCALE = 30.0
_B, _H, _W = 4096, 224, 224

_INFO = plsc.get_sparse_core_info()
_NC, _NS, _L = _INFO.num_cores, _INFO.num_subcores, _INFO.num_lanes
_NW = _NC * _NS
_BPW = _B // _NW
_CHUNKS = _BPW // _L

_mesh = plsc.VectorSubcoreMesh(core_axis_name="c", subcore_axis_name="s")


@functools.partial(
    pl.kernel,
    mesh=_mesh,
    compiler_params=pltpu.CompilerParams(needs_layout_passes=False),
    out_type=jax.ShapeDtypeStruct((_B,), jnp.float32),
    scratch_types=[
        pltpu.VMEM((_BPW,), jnp.int32),
        pltpu.VMEM((_BPW,), jnp.int32),
        pltpu.VMEM((_BPW,), jnp.int32),
        pltpu.VMEM((_BPW, _BPW), jnp.float32),
        pltpu.VMEM((_BPW,), jnp.float32),
        pltpu.SemaphoreType.DMA,
    ],
)
def _sc_gather(tab_hbm, x_hbm, y_hbm, out_hbm, x_v, y_v, row_v, sq_v,
               vals_v, sem):
    wid = lax.axis_index("s") * _NC + lax.axis_index("c")
    base = pl.multiple_of(wid * _BPW, _BPW)

    pltpu.sync_copy(x_hbm.at[pl.ds(base, _BPW)], x_v)
    pltpu.sync_copy(y_hbm.at[pl.ds(base, _BPW)], y_v)

    lane = lax.iota(jnp.int32, _L)
    for j in range(_CHUNKS):
        sl = pl.ds(_L * j, _L)
        row_v[sl] = y_v[sl] * _W + x_v[sl]

    pltpu.async_copy(
        tab_hbm.at[row_v, pl.ds(base, _BPW)], sq_v, sem).wait()

    for g in range(_CHUNKS):
        sl = pl.ds(_L * g, _L)
        iv = g * _L + lane
        picked = plsc.load_gather(sq_v, [iv, iv])
        vals_v[sl] = picked * _TIME_SCALE

    pltpu.sync_copy(vals_v, out_hbm.at[pl.ds(base, _BPW)])


def kernel(v_image, actor_pixel_selection):
    tab = jnp.transpose(v_image, (1, 2, 0)).reshape(_H * _W, _B)
    x = actor_pixel_selection[:, 0]
    y = actor_pixel_selection[:, 1]
    out = _sc_gather(tab, x, y)
    return out.reshape(_B, 1, 1)
